# baseline (speedup 1.0000x reference)
import functools

import jax
import jax.numpy as jnp
from jax import lax
from jax.experimental import pallas as pl
from jax.experimental.pallas import tpu as pltpu
from jax.experimental.pallas import tpu_sc as plsc

N = 50000
F = 64
HF = 32
T = 3
G = 16
E = 800000

BLK = 256
NP = 50176
NBLK = NP // BLK

NC = 2
NS = 16
SK = 512
EPT = 50176
EP = EPT * NS
NSUP = EPT // SK
SKC = SK // 128
ROWS_PER_TILE = NP // NS
ER = EP // 128


def _leaky(x):
    return jnp.where(x >= 0, x, 0.2 * x)


_GDN = lax.GatherDimensionNumbers(
    offset_dims=(), collapsed_slice_dims=(0,), start_index_map=(0,))


def _bcast_lane(vec, i):
    idx = jnp.full((16, 1), i, jnp.int32)
    return lax.gather(vec, idx, _GDN, (1,),
                      mode=lax.GatherScatterMode.PROMISE_IN_BOUNDS)


_MESH = plsc.VectorSubcoreMesh(core_axis_name="c", subcore_axis_name="s")


def _sc_body(ha, hb, src2d, dst2d, ew2d, zeros, outa, outb,
             src_v, dst_v, ew_v, rows_v, acc, isem, gsem, ssem):
    c = lax.axis_index("c")
    s = lax.axis_index("s")

    def run(h, out):
        zstart = s * ROWS_PER_TILE
        pltpu.async_copy(zeros, acc.at[pl.ds(zstart, ROWS_PER_TILE)], isem).wait()
        plsc.subcore_barrier()

        base_row = s * (EPT // 128)

        def drain_scatters():
            for j in range(SKC):
                pltpu.make_async_copy(rows_v.at[j], acc.at[dst_v.at[j]],
                                      ssem).wait()

        def super_body(sc, carry):
            @pl.when(sc > 0)
            def _():
                drain_scatters()

            row = base_row + sc * SKC
            d1 = pltpu.async_copy(src2d.at[pl.ds(row, SKC)], src_v, isem)
            d2 = pltpu.async_copy(dst2d.at[pl.ds(row, SKC)], dst_v, isem)
            d3 = pltpu.async_copy(ew2d.at[pl.ds(row, SKC)], ew_v, isem)
            d1.wait()
            d2.wait()
            d3.wait()
            gds = [pltpu.async_copy(h.at[src_v.at[j]], rows_v.at[j], gsem)
                   for j in range(SKC)]
            for j in range(SKC):
                gds[j].wait()

                def grp(g, _, j=j):
                    ew_vec = ew_v[j, pl.ds(g * 16, 16)]
                    for i in range(16):
                        e = g * 16 + i
                        scale = _bcast_lane(ew_vec, i)
                        r0 = rows_v[j, e, pl.ds(0, 16)]
                        rows_v[j, e, pl.ds(0, 16)] = r0 * scale
                        r1 = rows_v[j, e, pl.ds(16, 16)]
                        rows_v[j, e, pl.ds(16, 16)] = r1 * scale
                    return 0

                lax.fori_loop(0, 8, grp, 0)
                pltpu.async_copy(rows_v.at[j], acc.at[dst_v.at[j]],
                                 ssem, add=True)
            return carry

        lax.fori_loop(0, NSUP, super_body, 0)
        drain_scatters()
        plsc.subcore_barrier()
        pltpu.async_copy(acc.at[pl.ds(zstart, ROWS_PER_TILE)],
                         out.at[pl.ds(zstart, ROWS_PER_TILE)], isem).wait()

    @pl.when(c == 0)
    def _():
        run(ha, outa)

    @pl.when(c == 1)
    def _():
        run(hb, outb)


@functools.partial(
    pl.kernel,
    out_type=(jax.ShapeDtypeStruct((NP, HF), jnp.float32),
              jax.ShapeDtypeStruct((NP, HF), jnp.float32)),
    mesh=_MESH,
    scratch_types=[
        pltpu.VMEM((SKC, 128), jnp.int32),
        pltpu.VMEM((SKC, 128), jnp.int32),
        pltpu.VMEM((SKC, 128), jnp.float32),
        pltpu.VMEM((SKC, 128, HF), jnp.float32),
        pltpu.VMEM_SHARED((NP, HF), jnp.float32),
        pltpu.SemaphoreType.DMA,
        pltpu.SemaphoreType.DMA,
        pltpu.SemaphoreType.DMA,
    ],
    compiler_params=pltpu.CompilerParams(use_tc_tiling_on_sc=False),
)
def _sc_scatter(ha, hb, src2d, dst2d, ew2d, zeros, outa, outb,
                src_v, dst_v, ew_v, rows_v, acc, isem, gsem, ssem):
    _sc_body(ha, hb, src2d, dst2d, ew2d, zeros, outa, outb,
             src_v, dst_v, ew_v, rows_v, acc, isem, gsem, ssem)


def _dot(a, b):
    return jnp.dot(a, b, preferred_element_type=jnp.float32)


def _layer_first(x_pad, states_col, w1, b1, w2, b2, a0):
    def body(x_ref, st_ref, w1_ref, b1_ref, w2_ref, b2_ref, a0_ref,
             oa_ref, ob_ref):
        xi = x_ref[...]
        st = st_ref[...]
        t = _leaky(_dot(xi + a0_ref[0, 0] * st, w1_ref[...]) + b1_ref[...])
        h = _dot(t, w2_ref[...]) + b2_ref[...]
        oa_ref[...] = h[:, :HF]
        ob_ref[...] = h[:, HF:]

    return pl.pallas_call(
        body,
        grid=(NBLK,),
        in_specs=[
            pl.BlockSpec((BLK, F), lambda i: (i, 0)),
            pl.BlockSpec((BLK, 1), lambda i: (i, 0)),
            pl.BlockSpec((F, F), lambda i: (0, 0)),
            pl.BlockSpec((1, F), lambda i: (0, 0)),
            pl.BlockSpec((F, F), lambda i: (0, 0)),
            pl.BlockSpec((1, F), lambda i: (0, 0)),
            pl.BlockSpec((1, 1), lambda i: (0, 0)),
        ],
        out_specs=[
            pl.BlockSpec((BLK, HF), lambda i: (i, 0)),
            pl.BlockSpec((BLK, HF), lambda i: (i, 0)),
        ],
        out_shape=[
            jax.ShapeDtypeStruct((NP, HF), jnp.float32),
            jax.ShapeDtypeStruct((NP, HF), jnp.float32),
        ],
    )(x_pad, states_col, w1, b1, w2, b2, a0)


def _layer_next(sa, sb, states_col, w1, b1, w2, b2, a0):
    def body(sa_ref, sb_ref, st_ref, w1_ref, b1_ref, w2_ref, b2_ref, a0_ref,
             oa_ref, ob_ref):
        xi = _leaky(jnp.concatenate([sa_ref[...], sb_ref[...]], axis=1))
        st = st_ref[...]
        t = _leaky(_dot(xi + a0_ref[0, 0] * st, w1_ref[...]) + b1_ref[...])
        h = _dot(t, w2_ref[...]) + b2_ref[...]
        oa_ref[...] = h[:, :HF]
        ob_ref[...] = h[:, HF:]

    return pl.pallas_call(
        body,
        grid=(NBLK,),
        in_specs=[
            pl.BlockSpec((BLK, HF), lambda i: (i, 0)),
            pl.BlockSpec((BLK, HF), lambda i: (i, 0)),
            pl.BlockSpec((BLK, 1), lambda i: (i, 0)),
            pl.BlockSpec((F, F), lambda i: (0, 0)),
            pl.BlockSpec((1, F), lambda i: (0, 0)),
            pl.BlockSpec((F, F), lambda i: (0, 0)),
            pl.BlockSpec((1, F), lambda i: (0, 0)),
            pl.BlockSpec((1, 1), lambda i: (0, 0)),
        ],
        out_specs=[
            pl.BlockSpec((BLK, HF), lambda i: (i, 0)),
            pl.BlockSpec((BLK, HF), lambda i: (i, 0)),
        ],
        out_shape=[
            jax.ShapeDtypeStruct((NP, HF), jnp.float32),
            jax.ShapeDtypeStruct((NP, HF), jnp.float32),
        ],
    )(sa, sb, states_col, w1, b1, w2, b2, a0)


def _final_pool(s1, s2, states_col, batch_col, b0w, b1w, b2w, b2b):
    def body(s1a0, s1b0, s1a1, s1b1, s1a2, s1b2,
             s2a0, s2b0, s2a1, s2b1, s2a2, s2b2,
             st_ref, bt_ref, b0w_ref, b1w_ref, b2w_ref, b2b_ref,
             xc_ref, segm_ref, segall_ref):
        def xsum(refs):
            acc = None
            for (ra, rb) in refs:
                xi = _leaky(jnp.concatenate([ra[...], rb[...]], axis=1))
                acc = xi if acc is None else acc + xi
            return acc

        x1s = xsum([(s1a0, s1b0), (s1a1, s1b1), (s1a2, s1b2)])
        x2s = xsum([(s2a0, s2b0), (s2a1, s2b1), (s2a2, s2b2)])
        u = _dot(_dot(x1s, b0w_ref[...]), b2w_ref[:F, :])
        v = _dot(_dot(x2s, b1w_ref[...]), b2w_ref[F:, :])
        xc = _leaky(u + v + b2b_ref[...])
        xc_ref[...] = xc

        i = pl.program_id(0)

        @pl.when(i == 0)
        def _():
            segm_ref[...] = jnp.zeros_like(segm_ref)
            segall_ref[...] = jnp.zeros_like(segall_ref)

        bt = bt_ref[...]
        oh = (bt == lax.broadcasted_iota(jnp.int32, (1, G), 1)).astype(jnp.float32)
        mask = (st_ref[...] == 1.0).astype(jnp.float32)
        segall_ref[...] += lax.dot_general(
            oh, xc, (((0,), (0,)), ((), ())),
            preferred_element_type=jnp.float32)
        segm_ref[...] += lax.dot_general(
            oh, xc * mask, (((0,), (0,)), ((), ())),
            preferred_element_type=jnp.float32)

    nf = pl.BlockSpec((BLK, HF), lambda i: (i, 0))
    wf = pl.BlockSpec((F, F), lambda i: (0, 0))
    return pl.pallas_call(
        body,
        grid=(NBLK,),
        in_specs=[nf] * 12 + [
            pl.BlockSpec((BLK, 1), lambda i: (i, 0)),
            pl.BlockSpec((BLK, 1), lambda i: (i, 0)),
            wf, wf,
            pl.BlockSpec((2 * F, F), lambda i: (0, 0)),
            pl.BlockSpec((1, F), lambda i: (0, 0)),
        ],
        out_specs=[
            pl.BlockSpec((BLK, F), lambda i: (i, 0)),
            pl.BlockSpec((G, F), lambda i: (0, 0)),
            pl.BlockSpec((G, F), lambda i: (0, 0)),
        ],
        out_shape=[
            jax.ShapeDtypeStruct((NP, F), jnp.float32),
            jax.ShapeDtypeStruct((G, F), jnp.float32),
            jax.ShapeDtypeStruct((G, F), jnp.float32),
        ],
    )(*s1, *s2, states_col, batch_col, b0w, b1w, b2w, b2b)


def _graph_proj(segm, segall, g1w, g2w, g3w):
    def body(m_ref, a_ref, g1_ref, g2_ref, g3_ref, p_ref):
        p_ref[...] = (_dot(_dot(m_ref[...], g1_ref[...]), g3_ref[F:2 * F, :])
                      + _dot(_dot(a_ref[...], g2_ref[...]), g3_ref[2 * F:, :]))

    return pl.pallas_call(
        body,
        out_shape=jax.ShapeDtypeStruct((G, 3 * F // 2), jnp.float32),
    )(segm, segall, g1w, g2w, g3w)


def _final_out(xc, batch_col, p, g0w, g3w, g4w, g4b):
    def body(xc_ref, bt_ref, p_ref, g0_ref, g3_ref, g4w_ref, g4b_ref, o_ref):
        xc = xc_ref[...]
        bt = bt_ref[...]
        oh = (bt == lax.broadcasted_iota(jnp.int32, (1, G), 1)).astype(jnp.float32)
        xg = _dot(_dot(xc, g0_ref[...]), g3_ref[:F, :]) + _dot(oh, p_ref[...])
        xg = _leaky(xg)
        o_ref[...] = _dot(xg, g4w_ref[...]) + g4b_ref[0, 0]

    return pl.pallas_call(
        body,
        grid=(NBLK,),
        in_specs=[
            pl.BlockSpec((BLK, F), lambda i: (i, 0)),
            pl.BlockSpec((BLK, 1), lambda i: (i, 0)),
            pl.BlockSpec((G, 3 * F // 2), lambda i: (0, 0)),
            pl.BlockSpec((F, F), lambda i: (0, 0)),
            pl.BlockSpec((3 * F, 3 * F // 2), lambda i: (0, 0)),
            pl.BlockSpec((3 * F // 2, 1), lambda i: (0, 0)),
            pl.BlockSpec((1, 1), lambda i: (0, 0)),
        ],
        out_specs=pl.BlockSpec((BLK, 1), lambda i: (i, 0)),
        out_shape=jax.ShapeDtypeStruct((NP, 1), jnp.float32),
    )(xc, batch_col, p, g0w, g3w, g4w, g4b)


def kernel(x, edge_index, edge_weight, batch, states, params):
    f32 = jnp.float32
    x1_pad = jnp.zeros((NP, F), f32).at[:N].set(x[:, :F])
    x2_pad = jnp.zeros((NP, F), f32).at[:N].set(x[:, F:])
    states_col = jnp.zeros((NP, 1), f32).at[:N, 0].set(states)
    batch_col = jnp.full((NP, 1), G, jnp.int32).at[:N, 0].set(batch)

    a2d = jnp.zeros((EP,), jnp.int32).at[:E].set(edge_index[0]).reshape(ER, 128)
    b2d = jnp.zeros((EP,), jnp.int32).at[:E].set(edge_index[1]).reshape(ER, 128)
    ew2d = jnp.zeros((EP,), f32).at[:E].set(edge_weight).reshape(ER, 128)
    zeros_tbl = jnp.zeros((ROWS_PER_TILE, HF), f32)

    def wts(p):
        return (p["alpha1"]["W"], p["alpha1"]["b"].reshape(1, F),
                p["lin"]["W"], p["lin"]["b"].reshape(1, F),
                p["alpha0"]["W"].reshape(1, 1))

    h1 = _layer_first(x1_pad, states_col, *wts(params["blocks1"][0]))
    h2 = _layer_first(x2_pad, states_col, *wts(params["blocks2"][0]))

    s1_list, s2_list = [], []
    for t in range(T):
        s1 = _sc_scatter(h1[0], h1[1], b2d, a2d, ew2d, zeros_tbl)
        s2 = _sc_scatter(h2[0], h2[1], a2d, b2d, ew2d, zeros_tbl)
        s1_list.append(s1)
        s2_list.append(s2)
        if t + 1 < T:
            h1 = _layer_next(s1[0], s1[1], states_col,
                             *wts(params["blocks1"][t + 1]))
            h2 = _layer_next(s2[0], s2[1], states_col,
                             *wts(params["blocks2"][t + 1]))

    s1_flat = [r for s in s1_list for r in s]
    s2_flat = [r for s in s2_list for r in s]
    xc, segm, segall = _final_pool(
        s1_flat, s2_flat, states_col, batch_col,
        params["beta0"]["W"], params["beta1"]["W"], params["beta2"]["W"],
        params["beta2"]["b"].reshape(1, F))
    p = _graph_proj(segm, segall, params["gamma1"]["W"], params["gamma2"]["W"],
                    params["gamma3"]["W"])
    out = _final_out(xc, batch_col, p, params["gamma0"]["W"],
                     params["gamma3"]["W"], params["gamma4"]["W"],
                     params["gamma4"]["b"].reshape(1, 1))
    return out[:N, 0]


# speedup vs baseline: 1.1487x; 1.1487x over previous
import functools

import jax
import jax.numpy as jnp
from jax import lax
from jax.experimental import pallas as pl
from jax.experimental.pallas import tpu as pltpu
from jax.experimental.pallas import tpu_sc as plsc

N = 50000
F = 64
HF = 32
T = 3
G = 16
E = 800000

BLK = 256
NP = 50176
NBLK = NP // BLK

NC = 2
NS = 16
SK = 512
EPT = 50176
EP = EPT * NS
NSUP = EPT // SK
SKC = SK // 128
ROWS_PER_TILE = NP // NS
ER = EP // 128


def _leaky(x):
    return jnp.where(x >= 0, x, 0.2 * x)


_GDN = lax.GatherDimensionNumbers(
    offset_dims=(), collapsed_slice_dims=(0,), start_index_map=(0,))


def _bcast_lane(vec, i):
    idx = jnp.full((16, 1), i, jnp.int32)
    return lax.gather(vec, idx, _GDN, (1,),
                      mode=lax.GatherScatterMode.PROMISE_IN_BOUNDS)


_MESH = plsc.VectorSubcoreMesh(core_axis_name="c", subcore_axis_name="s")


def _sc_body(ha, hb, src2d, dst2d, ew2d, zeros, outa, outb,
             src_a, dst_a, ew_a, src_b, dst_b, ew_b,
             rows_v, acc, isem, gsem, ssem):
    c = lax.axis_index("c")
    s = lax.axis_index("s")
    seta = (src_a, dst_a, ew_a)
    setb = (src_b, dst_b, ew_b)

    def run(h, out):
        zstart = s * ROWS_PER_TILE
        pltpu.async_copy(zeros, acc.at[pl.ds(zstart, ROWS_PER_TILE)], isem).wait()
        plsc.subcore_barrier()

        base_row = s * (EPT // 128)

        def idx_issue(sup, st):
            row = base_row + sup * SKC
            pltpu.async_copy(src2d.at[pl.ds(row, SKC)], st[0], isem)
            pltpu.async_copy(dst2d.at[pl.ds(row, SKC)], st[1], isem)
            pltpu.async_copy(ew2d.at[pl.ds(row, SKC)], st[2], isem)

        def idx_wait(st):
            pltpu.make_async_copy(src2d.at[pl.ds(0, SKC)], st[0], isem).wait()
            pltpu.make_async_copy(dst2d.at[pl.ds(0, SKC)], st[1], isem).wait()
            pltpu.make_async_copy(ew2d.at[pl.ds(0, SKC)], st[2], isem).wait()

        def gather_issue(st, j):
            pltpu.async_copy(h.at[st[0].at[j]], rows_v.at[j], gsem)

        def gather_wait(st, j):
            pltpu.make_async_copy(h.at[st[0].at[j]], rows_v.at[j], gsem).wait()

        def scatter_issue(st, j):
            pltpu.async_copy(rows_v.at[j], acc.at[st[1].at[j]], ssem, add=True)

        def scatter_wait(st, j):
            pltpu.make_async_copy(rows_v.at[j], acc.at[st[1].at[j]], ssem).wait()

        def compute(st, j):
            def grp(g, _, j=j):
                ew_vec = st[2][j, pl.ds(g * 16, 16)]
                for i in range(16):
                    e = g * 16 + i
                    scale = _bcast_lane(ew_vec, i)
                    r0 = rows_v[j, e, pl.ds(0, 16)]
                    rows_v[j, e, pl.ds(0, 16)] = r0 * scale
                    r1 = rows_v[j, e, pl.ds(16, 16)]
                    rows_v[j, e, pl.ds(16, 16)] = r1 * scale
                return 0

            lax.fori_loop(0, 8, grp, 0)

        def slot(sup, my, other, first, last):
            if not first:
                scatter_wait(other, SKC - 1)
            if not last:
                idx_issue(sup + 1, other)
            gather_issue(my, SKC - 1)
            for j in range(SKC):
                gather_wait(my, j)
                compute(my, j)
                scatter_issue(my, j)
            for j in range(SKC - 1):
                scatter_wait(my, j)
            if not last:
                idx_wait(other)
                for j in range(SKC - 1):
                    gather_issue(other, j)

        idx_issue(0, seta)
        idx_wait(seta)
        for j in range(SKC - 1):
            gather_issue(seta, j)

        slot(0, seta, setb, True, False)

        def iter_k(k, carry):
            s0 = 2 * k + 1
            slot(s0, setb, seta, False, False)
            slot(s0 + 1, seta, setb, False, False)
            return carry

        lax.fori_loop(0, (NSUP - 2) // 2, iter_k, 0)
        slot(NSUP - 1, setb, seta, False, True)
        scatter_wait(setb, SKC - 1)
        plsc.subcore_barrier()
        pltpu.async_copy(acc.at[pl.ds(zstart, ROWS_PER_TILE)],
                         out.at[pl.ds(zstart, ROWS_PER_TILE)], isem).wait()

    @pl.when(c == 0)
    def _():
        run(ha, outa)

    @pl.when(c == 1)
    def _():
        run(hb, outb)


@functools.partial(
    pl.kernel,
    out_type=(jax.ShapeDtypeStruct((NP, HF), jnp.float32),
              jax.ShapeDtypeStruct((NP, HF), jnp.float32)),
    mesh=_MESH,
    scratch_types=[
        pltpu.VMEM((SKC, 128), jnp.int32),
        pltpu.VMEM((SKC, 128), jnp.int32),
        pltpu.VMEM((SKC, 128), jnp.float32),
        pltpu.VMEM((SKC, 128), jnp.int32),
        pltpu.VMEM((SKC, 128), jnp.int32),
        pltpu.VMEM((SKC, 128), jnp.float32),
        pltpu.VMEM((SKC, 128, HF), jnp.float32),
        pltpu.VMEM_SHARED((NP, HF), jnp.float32),
        pltpu.SemaphoreType.DMA,
        pltpu.SemaphoreType.DMA,
        pltpu.SemaphoreType.DMA,
    ],
    compiler_params=pltpu.CompilerParams(use_tc_tiling_on_sc=False),
)
def _sc_scatter(ha, hb, src2d, dst2d, ew2d, zeros, outa, outb,
                src_a, dst_a, ew_a, src_b, dst_b, ew_b,
                rows_v, acc, isem, gsem, ssem):
    _sc_body(ha, hb, src2d, dst2d, ew2d, zeros, outa, outb,
             src_a, dst_a, ew_a, src_b, dst_b, ew_b,
             rows_v, acc, isem, gsem, ssem)


def _dot(a, b):
    return jnp.dot(a, b, preferred_element_type=jnp.float32)


def _layer_first(x_pad, states_col, w1, b1, w2, b2, a0):
    def body(x_ref, st_ref, w1_ref, b1_ref, w2_ref, b2_ref, a0_ref,
             oa_ref, ob_ref):
        xi = x_ref[...]
        st = st_ref[...]
        t = _leaky(_dot(xi + a0_ref[0, 0] * st, w1_ref[...]) + b1_ref[...])
        h = _dot(t, w2_ref[...]) + b2_ref[...]
        oa_ref[...] = h[:, :HF]
        ob_ref[...] = h[:, HF:]

    return pl.pallas_call(
        body,
        grid=(NBLK,),
        in_specs=[
            pl.BlockSpec((BLK, F), lambda i: (i, 0)),
            pl.BlockSpec((BLK, 1), lambda i: (i, 0)),
            pl.BlockSpec((F, F), lambda i: (0, 0)),
            pl.BlockSpec((1, F), lambda i: (0, 0)),
            pl.BlockSpec((F, F), lambda i: (0, 0)),
            pl.BlockSpec((1, F), lambda i: (0, 0)),
            pl.BlockSpec((1, 1), lambda i: (0, 0)),
        ],
        out_specs=[
            pl.BlockSpec((BLK, HF), lambda i: (i, 0)),
            pl.BlockSpec((BLK, HF), lambda i: (i, 0)),
        ],
        out_shape=[
            jax.ShapeDtypeStruct((NP, HF), jnp.float32),
            jax.ShapeDtypeStruct((NP, HF), jnp.float32),
        ],
    )(x_pad, states_col, w1, b1, w2, b2, a0)


def _layer_next(sa, sb, states_col, w1, b1, w2, b2, a0):
    def body(sa_ref, sb_ref, st_ref, w1_ref, b1_ref, w2_ref, b2_ref, a0_ref,
             oa_ref, ob_ref):
        xi = _leaky(jnp.concatenate([sa_ref[...], sb_ref[...]], axis=1))
        st = st_ref[...]
        t = _leaky(_dot(xi + a0_ref[0, 0] * st, w1_ref[...]) + b1_ref[...])
        h = _dot(t, w2_ref[...]) + b2_ref[...]
        oa_ref[...] = h[:, :HF]
        ob_ref[...] = h[:, HF:]

    return pl.pallas_call(
        body,
        grid=(NBLK,),
        in_specs=[
            pl.BlockSpec((BLK, HF), lambda i: (i, 0)),
            pl.BlockSpec((BLK, HF), lambda i: (i, 0)),
            pl.BlockSpec((BLK, 1), lambda i: (i, 0)),
            pl.BlockSpec((F, F), lambda i: (0, 0)),
            pl.BlockSpec((1, F), lambda i: (0, 0)),
            pl.BlockSpec((F, F), lambda i: (0, 0)),
            pl.BlockSpec((1, F), lambda i: (0, 0)),
            pl.BlockSpec((1, 1), lambda i: (0, 0)),
        ],
        out_specs=[
            pl.BlockSpec((BLK, HF), lambda i: (i, 0)),
            pl.BlockSpec((BLK, HF), lambda i: (i, 0)),
        ],
        out_shape=[
            jax.ShapeDtypeStruct((NP, HF), jnp.float32),
            jax.ShapeDtypeStruct((NP, HF), jnp.float32),
        ],
    )(sa, sb, states_col, w1, b1, w2, b2, a0)


def _final_pool(s1, s2, states_col, batch_col, b0w, b1w, b2w, b2b):
    def body(s1a0, s1b0, s1a1, s1b1, s1a2, s1b2,
             s2a0, s2b0, s2a1, s2b1, s2a2, s2b2,
             st_ref, bt_ref, b0w_ref, b1w_ref, b2w_ref, b2b_ref,
             xc_ref, segm_ref, segall_ref):
        def xsum(refs):
            acc = None
            for (ra, rb) in refs:
                xi = _leaky(jnp.concatenate([ra[...], rb[...]], axis=1))
                acc = xi if acc is None else acc + xi
            return acc

        x1s = xsum([(s1a0, s1b0), (s1a1, s1b1), (s1a2, s1b2)])
        x2s = xsum([(s2a0, s2b0), (s2a1, s2b1), (s2a2, s2b2)])
        u = _dot(_dot(x1s, b0w_ref[...]), b2w_ref[:F, :])
        v = _dot(_dot(x2s, b1w_ref[...]), b2w_ref[F:, :])
        xc = _leaky(u + v + b2b_ref[...])
        xc_ref[...] = xc

        i = pl.program_id(0)

        @pl.when(i == 0)
        def _():
            segm_ref[...] = jnp.zeros_like(segm_ref)
            segall_ref[...] = jnp.zeros_like(segall_ref)

        bt = bt_ref[...]
        oh = (bt == lax.broadcasted_iota(jnp.int32, (1, G), 1)).astype(jnp.float32)
        mask = (st_ref[...] == 1.0).astype(jnp.float32)
        segall_ref[...] += lax.dot_general(
            oh, xc, (((0,), (0,)), ((), ())),
            preferred_element_type=jnp.float32)
        segm_ref[...] += lax.dot_general(
            oh, xc * mask, (((0,), (0,)), ((), ())),
            preferred_element_type=jnp.float32)

    nf = pl.BlockSpec((BLK, HF), lambda i: (i, 0))
    wf = pl.BlockSpec((F, F), lambda i: (0, 0))
    return pl.pallas_call(
        body,
        grid=(NBLK,),
        in_specs=[nf] * 12 + [
            pl.BlockSpec((BLK, 1), lambda i: (i, 0)),
            pl.BlockSpec((BLK, 1), lambda i: (i, 0)),
            wf, wf,
            pl.BlockSpec((2 * F, F), lambda i: (0, 0)),
            pl.BlockSpec((1, F), lambda i: (0, 0)),
        ],
        out_specs=[
            pl.BlockSpec((BLK, F), lambda i: (i, 0)),
            pl.BlockSpec((G, F), lambda i: (0, 0)),
            pl.BlockSpec((G, F), lambda i: (0, 0)),
        ],
        out_shape=[
            jax.ShapeDtypeStruct((NP, F), jnp.float32),
            jax.ShapeDtypeStruct((G, F), jnp.float32),
            jax.ShapeDtypeStruct((G, F), jnp.float32),
        ],
    )(*s1, *s2, states_col, batch_col, b0w, b1w, b2w, b2b)


def _graph_proj(segm, segall, g1w, g2w, g3w):
    def body(m_ref, a_ref, g1_ref, g2_ref, g3_ref, p_ref):
        p_ref[...] = (_dot(_dot(m_ref[...], g1_ref[...]), g3_ref[F:2 * F, :])
                      + _dot(_dot(a_ref[...], g2_ref[...]), g3_ref[2 * F:, :]))

    return pl.pallas_call(
        body,
        out_shape=jax.ShapeDtypeStruct((G, 3 * F // 2), jnp.float32),
    )(segm, segall, g1w, g2w, g3w)


def _final_out(xc, batch_col, p, g0w, g3w, g4w, g4b):
    def body(xc_ref, bt_ref, p_ref, g0_ref, g3_ref, g4w_ref, g4b_ref, o_ref):
        xc = xc_ref[...]
        bt = bt_ref[...]
        oh = (bt == lax.broadcasted_iota(jnp.int32, (1, G), 1)).astype(jnp.float32)
        xg = _dot(_dot(xc, g0_ref[...]), g3_ref[:F, :]) + _dot(oh, p_ref[...])
        xg = _leaky(xg)
        o_ref[...] = _dot(xg, g4w_ref[...]) + g4b_ref[0, 0]

    return pl.pallas_call(
        body,
        grid=(NBLK,),
        in_specs=[
            pl.BlockSpec((BLK, F), lambda i: (i, 0)),
            pl.BlockSpec((BLK, 1), lambda i: (i, 0)),
            pl.BlockSpec((G, 3 * F // 2), lambda i: (0, 0)),
            pl.BlockSpec((F, F), lambda i: (0, 0)),
            pl.BlockSpec((3 * F, 3 * F // 2), lambda i: (0, 0)),
            pl.BlockSpec((3 * F // 2, 1), lambda i: (0, 0)),
            pl.BlockSpec((1, 1), lambda i: (0, 0)),
        ],
        out_specs=pl.BlockSpec((BLK, 1), lambda i: (i, 0)),
        out_shape=jax.ShapeDtypeStruct((NP, 1), jnp.float32),
    )(xc, batch_col, p, g0w, g3w, g4w, g4b)


def kernel(x, edge_index, edge_weight, batch, states, params):
    f32 = jnp.float32
    x1_pad = jnp.zeros((NP, F), f32).at[:N].set(x[:, :F])
    x2_pad = jnp.zeros((NP, F), f32).at[:N].set(x[:, F:])
    states_col = jnp.zeros((NP, 1), f32).at[:N, 0].set(states)
    batch_col = jnp.full((NP, 1), G, jnp.int32).at[:N, 0].set(batch)

    a2d = jnp.zeros((EP,), jnp.int32).at[:E].set(edge_index[0]).reshape(ER, 128)
    b2d = jnp.zeros((EP,), jnp.int32).at[:E].set(edge_index[1]).reshape(ER, 128)
    ew2d = jnp.zeros((EP,), f32).at[:E].set(edge_weight).reshape(ER, 128)
    zeros_tbl = jnp.zeros((ROWS_PER_TILE, HF), f32)

    def wts(p):
        return (p["alpha1"]["W"], p["alpha1"]["b"].reshape(1, F),
                p["lin"]["W"], p["lin"]["b"].reshape(1, F),
                p["alpha0"]["W"].reshape(1, 1))

    h1 = _layer_first(x1_pad, states_col, *wts(params["blocks1"][0]))
    h2 = _layer_first(x2_pad, states_col, *wts(params["blocks2"][0]))

    s1_list, s2_list = [], []
    for t in range(T):
        s1 = _sc_scatter(h1[0], h1[1], b2d, a2d, ew2d, zeros_tbl)
        s2 = _sc_scatter(h2[0], h2[1], a2d, b2d, ew2d, zeros_tbl)
        s1_list.append(s1)
        s2_list.append(s2)
        if t + 1 < T:
            h1 = _layer_next(s1[0], s1[1], states_col,
                             *wts(params["blocks1"][t + 1]))
            h2 = _layer_next(s2[0], s2[1], states_col,
                             *wts(params["blocks2"][t + 1]))

    s1_flat = [r for s in s1_list for r in s]
    s2_flat = [r for s in s2_list for r in s]
    xc, segm, segall = _final_pool(
        s1_flat, s2_flat, states_col, batch_col,
        params["beta0"]["W"], params["beta1"]["W"], params["beta2"]["W"],
        params["beta2"]["b"].reshape(1, F))
    p = _graph_proj(segm, segall, params["gamma1"]["W"], params["gamma2"]["W"],
                    params["gamma3"]["W"])
    out = _final_out(xc, batch_col, p, params["gamma0"]["W"],
                     params["gamma3"]["W"], params["gamma4"]["W"],
                     params["gamma4"]["b"].reshape(1, 1))
    return out[:N, 0]
